# baseline (speedup 1.0000x reference)
import functools

import jax
import jax.numpy as jnp
import numpy as np
from jax import lax
from jax.experimental import pallas as pl
from jax.experimental.pallas import tpu as pltpu
from jax.experimental.pallas import tpu_sc as plsc

N = 10000
C = 128
E = 320000
NC = 2
NS = 16
NW = NC * NS
L = 16
EPT = E // NW
CH = 48
GROUPS = CH // L
CHUNKS = -(-EPT // CH)
EXT_W = C + 16
TBL_W = 160
ROWS_PT = 632
NPAD = ROWS_PT * NS
CNT_W = 16
CROWS_PT = 160
CPAD = CROWS_PT * NS

_f32 = jnp.float32
_i32 = jnp.int32

BR = 1000

_PERM = np.empty((C,), np.int32)
for _j in range(C // 32):
    for _t in range(16):
        _PERM[32 * _j + _t] = 32 * _j + 2 * _t
        _PERM[32 * _j + 16 + _t] = 32 * _j + 2 * _t + 1
_INV = np.argsort(_PERM).astype(np.int32)


def _linear_norm(x, w, b):
    h = lax.dot_general(x, w, (((1,), (1,)), ((), ())),
                        preferred_element_type=_f32) + b
    nrm = jnp.sqrt(jnp.sum(h * h, axis=1, keepdims=True))
    scale = jnp.maximum(nrm, 1e-12)
    norm = h / scale
    selfmsg = jnp.sum(norm * norm, axis=1, keepdims=True) * h
    hi = scale.astype(jnp.bfloat16)
    lo = (scale - hi.astype(_f32)).astype(jnp.bfloat16)
    pad = jnp.zeros((norm.shape[0], TBL_W - C - 2), jnp.bfloat16)
    tbl = jnp.concatenate([norm.astype(jnp.bfloat16), hi, lo, pad], axis=1)
    return tbl, selfmsg


def _pre_body(x_ref, w_ref, b_ref, norm_ref, self_ref):
    norm, selfmsg = _linear_norm(x_ref[...], w_ref[...], b_ref[...])
    norm_ref[...] = norm
    self_ref[...] = selfmsg


def _tc_pre(x, w, b):
    return pl.pallas_call(
        _pre_body,
        grid=(N // BR,),
        in_specs=[pl.BlockSpec((BR, C), lambda i: (i, 0)),
                  pl.BlockSpec((C, C), lambda i: (0, 0)),
                  pl.BlockSpec((1, C), lambda i: (0, 0))],
        out_specs=[pl.BlockSpec((BR, TBL_W), lambda i: (i, 0)),
                   pl.BlockSpec((BR, C), lambda i: (i, 0))],
        out_shape=[jax.ShapeDtypeStruct((N, TBL_W), jnp.bfloat16),
                   jax.ShapeDtypeStruct((N, C), _f32)],
    )(x, w, b.reshape(1, C))


def _combine(a0, a1, c0, c1, selfmsg, bias):
    summed = a0 + a1 + selfmsg
    cnt = c0[:, 0:1] + c1[:, 0:1] + 1.0
    return summed / jnp.maximum(cnt, 1.0) + bias


def _mid_body(a0_ref, a1_ref, c0_ref, c1_ref, self_ref, bias_ref, w_ref,
              b_ref, norm_ref, self2_ref):
    x2 = _combine(a0_ref[0], a1_ref[0], c0_ref[0], c1_ref[0],
                  self_ref[...], bias_ref[...])
    x2 = jnp.maximum(x2, 0.0)
    norm, selfmsg = _linear_norm(x2, w_ref[...], b_ref[...])
    norm_ref[...] = norm
    self2_ref[...] = selfmsg


def _tc_mid(acc, cnt, selfmsg, bias, w, b):
    return pl.pallas_call(
        _mid_body,
        grid=(N // BR,),
        in_specs=[pl.BlockSpec((1, BR, C), lambda i: (0, i, 0)),
                  pl.BlockSpec((1, BR, C), lambda i: (1, i, 0)),
                  pl.BlockSpec((1, BR, 4), lambda i: (0, i, 0)),
                  pl.BlockSpec((1, BR, 4), lambda i: (1, i, 0)),
                  pl.BlockSpec((BR, C), lambda i: (i, 0)),
                  pl.BlockSpec((1, C), lambda i: (0, 0)),
                  pl.BlockSpec((C, C), lambda i: (0, 0)),
                  pl.BlockSpec((1, C), lambda i: (0, 0))],
        out_specs=[pl.BlockSpec((BR, TBL_W), lambda i: (i, 0)),
                   pl.BlockSpec((BR, C), lambda i: (i, 0))],
        out_shape=[jax.ShapeDtypeStruct((N, TBL_W), jnp.bfloat16),
                   jax.ShapeDtypeStruct((N, C), _f32)],
    )(acc, acc, cnt, cnt, selfmsg, bias.reshape(1, C), w, b.reshape(1, C))


def _final_body(a0_ref, a1_ref, c0_ref, c1_ref, self_ref, bias_ref,
                out_ref):
    h = _combine(a0_ref[0], a1_ref[0], c0_ref[0], c1_ref[0],
                 self_ref[...], bias_ref[...])
    m = jnp.max(h, axis=1, keepdims=True)
    z = h - m
    out_ref[...] = z - jnp.log(jnp.sum(jnp.exp(z), axis=1, keepdims=True))


def _tc_final(acc, cnt, selfmsg, bias):
    return pl.pallas_call(
        _final_body,
        grid=(N // BR,),
        in_specs=[pl.BlockSpec((1, BR, C), lambda i: (0, i, 0)),
                  pl.BlockSpec((1, BR, C), lambda i: (1, i, 0)),
                  pl.BlockSpec((1, BR, 4), lambda i: (0, i, 0)),
                  pl.BlockSpec((1, BR, 4), lambda i: (1, i, 0)),
                  pl.BlockSpec((BR, C), lambda i: (i, 0)),
                  pl.BlockSpec((1, C), lambda i: (0, 0))],
        out_specs=pl.BlockSpec((BR, C), lambda i: (i, 0)),
        out_shape=jax.ShapeDtypeStruct((N, C), _f32),
    )(acc, acc, cnt, cnt, selfmsg, bias.reshape(1, C))


def _sc_edge_body(edge_ref, norm_ref, zeros_ref, zeros_cnt_ref,
                  out_ref, cnt_out_ref,
                  sidx, didx, sdidx, didx2, nsrc, ndst, msg, cntbuf,
                  acc_sh, cnt_sh, isem, gsem, ssem):
    cc = lax.axis_index("c")
    ss = lax.axis_index("s")
    wid = cc * NS + ss

    pltpu.sync_copy(zeros_ref.at[pl.ds(ss * ROWS_PT, ROWS_PT)],
                    acc_sh.at[pl.ds(ss * ROWS_PT, ROWS_PT)])
    pltpu.sync_copy(zeros_cnt_ref.at[pl.ds(ss * CROWS_PT, CROWS_PT)],
                    cnt_sh.at[pl.ds(ss * CROWS_PT, CROWS_PT)])

    iota16 = lax.iota(_i32, L)
    ones16 = jnp.ones((L,), _f32)
    zeros16 = jnp.zeros((L,), _f32)
    for S in range(2):
        for g in range(GROUPS):
            e16 = iota16 + (g * L)
            for col in range(CNT_W):
                plsc.store_scatter(cntbuf[S],
                                   [e16, jnp.full((L,), col, _i32)], zeros16)

    plsc.subcore_barrier()

    base0 = wid * EPT

    def issue_idx(k, S):
        base = base0 + jnp.minimum(k * CH, EPT - CH)
        pltpu.async_copy(edge_ref.at[pl.ds(base, CH)], sidx[S], isem[S])
        pltpu.async_copy(edge_ref.at[pl.ds(E + base, CH)], didx[S], isem[S])

    def wait_idx(S):
        pltpu.make_async_copy(edge_ref.at[pl.ds(0, CH)], sidx[S],
                              isem[S]).wait()
        pltpu.make_async_copy(edge_ref.at[pl.ds(0, CH)], didx[S],
                              isem[S]).wait()

    def issue_gather(S):
        pltpu.async_copy(norm_ref.at[sidx[S]], nsrc[S], gsem[S])
        pltpu.async_copy(norm_ref.at[didx[S]], ndst[S], gsem[S])

    def wait_gather(S):
        pltpu.make_async_copy(norm_ref.at[sidx[S]], nsrc[S], gsem[S]).wait()
        pltpu.make_async_copy(norm_ref.at[didx[S]], ndst[S], gsem[S]).wait()

    def issue_scatter(S):
        pltpu.async_copy(msg[S], acc_sh.at[sdidx[S]], ssem[S], add=True)
        pltpu.async_copy(cntbuf[S], cnt_sh.at[didx2[S]], ssem[S], add=True)

    def wait_scatter(S):
        pltpu.make_async_copy(msg[S], acc_sh.at[sdidx[S]], ssem[S]).wait()
        pltpu.make_async_copy(cntbuf[S], cnt_sh.at[didx2[S]], ssem[S]).wait()

    def dst_save(S4, S2, dummies=()):
        for g in range(GROUPS):
            e16 = iota16 + (g * L)
            dst16 = didx[S4][pl.ds(g * L, L)]
            if g in dummies:
                dst16 = jnp.full((L,), N, _i32)
            sdidx[S2][pl.ds(g * L, L)] = dst16
            didx2[S2][pl.ds(g * L, L)] = lax.shift_right_logical(dst16, 2)
            q = jnp.bitwise_and(dst16, 3)
            for i in range(4):
                col = lax.shift_left(jnp.bitwise_xor(q, i), 2)
                plsc.store_scatter(cntbuf[S2], [e16, col],
                                   ones16 if i == 0 else zeros16)

    def dot_mul(S4, S2):
        def edge_body(e):
            sab = [plsc.unpack(nsrc[S4][e, pl.ds(32 * j, 32)],
                               format=plsc.PackFormat.INTERLEAVED)
                   for j in range(4)]
            dab = [plsc.unpack(ndst[S4][e, pl.ds(32 * j, 32)],
                               format=plsc.PackFormat.INTERLEAVED)
                   for j in range(4)]
            m = [sab[j][0] * dab[j][0] + sab[j][1] * dab[j][1]
                 for j in range(4)]
            ha, hb = plsc.unpack(nsrc[S4][e, pl.ds(C, 32)],
                                 format=plsc.PackFormat.INTERLEAVED)
            sc = ha[0] + hb[0]
            coef = jnp.sum((m[0] + m[1]) + (m[2] + m[3])) * sc
            for j in range(4):
                msg[S2][e, pl.ds(32 * j, L)] = sab[j][0] * coef
                msg[S2][e, pl.ds(32 * j + L, L)] = sab[j][1] * coef

        plsc.parallel_loop(0, CH, unroll=2)(edge_body)

    def step(k, S4, pf_idx, pf_gather, wait_scat, dummies=()):
        S2 = S4 % 2
        wait_gather(S4)
        if pf_gather:
            wait_idx((S4 + 2) % 4)
            issue_gather((S4 + 2) % 4)
        if wait_scat:
            wait_scatter(S2)
        dst_save(S4, S2, dummies)
        if pf_idx:
            issue_idx(k + 4, S4)
        dot_mul(S4, S2)
        issue_scatter(S2)

    for j in range(4):
        issue_idx(j, j)
    wait_idx(0)
    issue_gather(0)
    wait_idx(1)
    issue_gather(1)
    step(0, 0, True, True, False)
    step(1, 1, True, True, False)
    step(2, 2, True, True, True)
    step(3, 3, True, True, True)

    def quad(kk, carry):
        k0 = 4 * kk
        for j in range(4):
            step(k0 + j, j, True, True, True)
        return carry

    lax.fori_loop(1, (CHUNKS - 9) // 4 + 1, quad, 0)

    step(CHUNKS - 5, 0, True, True, True)
    step(CHUNKS - 4, 1, False, True, True)
    step(CHUNKS - 3, 2, False, True, True)
    step(CHUNKS - 2, 3, False, False, True)
    step(CHUNKS - 1, 0, False, False, True,
         dummies=tuple(range(GROUPS - (EPT - (CHUNKS - 1) * CH) // L)))
    wait_scatter(1)
    wait_scatter(0)

    plsc.subcore_barrier()
    pltpu.sync_copy(acc_sh.at[pl.ds(ss * ROWS_PT, ROWS_PT)],
                    out_ref.at[cc, pl.ds(ss * ROWS_PT, ROWS_PT)])
    pltpu.sync_copy(cnt_sh.at[pl.ds(ss * CROWS_PT, CROWS_PT)],
                    cnt_out_ref.at[cc, pl.ds(ss * CROWS_PT, CROWS_PT)])


_sc_mesh = plsc.VectorSubcoreMesh(core_axis_name="c", subcore_axis_name="s",
                                  num_cores=NC, num_subcores=NS)

_sc_edge = functools.partial(
    pl.kernel,
    out_type=(jax.ShapeDtypeStruct((NC, NPAD, C), _f32),
              jax.ShapeDtypeStruct((NC, CPAD, CNT_W), _f32)),
    mesh=_sc_mesh,
    compiler_params=pltpu.CompilerParams(needs_layout_passes=False,
                                         use_tc_tiling_on_sc=False),
    scratch_types=[
        [pltpu.VMEM((CH,), _i32)] * 4,
        [pltpu.VMEM((CH,), _i32)] * 4,
        [pltpu.VMEM((CH,), _i32)] * 2,
        [pltpu.VMEM((CH,), _i32)] * 2,
        [pltpu.VMEM((CH, TBL_W), jnp.bfloat16)] * 4,
        [pltpu.VMEM((CH, TBL_W), jnp.bfloat16)] * 4,
        [pltpu.VMEM((CH, C), _f32)] * 2,
        [pltpu.VMEM((CH, CNT_W), _f32)] * 2,
        pltpu.MemorySpace.VMEM_SHARED((NPAD, C), _f32),
        pltpu.MemorySpace.VMEM_SHARED((CPAD, CNT_W), _f32),
        [pltpu.SemaphoreType.DMA] * 4,
        [pltpu.SemaphoreType.DMA] * 4,
        [pltpu.SemaphoreType.DMA] * 2,
    ],
)(_sc_edge_body)


def kernel(x, edge_index, W1, b1, bias1, W2, b2, bias2):
    zeros = jnp.zeros((NPAD, C), _f32)
    zeros_cnt = jnp.zeros((CPAD, CNT_W), _f32)
    edge_flat = edge_index.reshape(2 * E)
    inv = jnp.asarray(_INV)
    norm1, self1 = _tc_pre(x, W1, b1)
    acc1, cnt1 = _sc_edge(edge_flat, norm1, zeros, zeros_cnt)
    acc1 = jnp.take(acc1, inv, axis=2)
    cnt1 = cnt1.reshape(NC, CPAD * 4, 4)
    norm2, self2 = _tc_mid(acc1, cnt1, self1, bias1, W2, b2)
    acc2, cnt2 = _sc_edge(edge_flat, norm2, zeros, zeros_cnt)
    acc2 = jnp.take(acc2, inv, axis=2)
    cnt2 = cnt2.reshape(NC, CPAD * 4, 4)
    return _tc_final(acc2, cnt2, self2, bias2)


# speedup vs baseline: 1.1583x; 1.1583x over previous
import functools

import jax
import jax.numpy as jnp
import numpy as np
from jax import lax
from jax.experimental import pallas as pl
from jax.experimental.pallas import tpu as pltpu
from jax.experimental.pallas import tpu_sc as plsc

N = 10000
C = 128
E = 320000
NC = 2
NS = 16
NW = NC * NS
L = 16
EPT = E // NW
CH = 48
GROUPS = CH // L
CHUNKS = -(-EPT // CH)
EXT_W = C + 16
TBL_W = 160
ROWS_PT = 632
NPAD = ROWS_PT * NS
CNT_W = 16
CROWS_PT = 160
CPAD = CROWS_PT * NS

_f32 = jnp.float32
_i32 = jnp.int32

BR = 1000

_PERM = np.empty((C,), np.int32)
for _j in range(C // 32):
    for _t in range(16):
        _PERM[32 * _j + _t] = 32 * _j + 2 * _t
        _PERM[32 * _j + 16 + _t] = 32 * _j + 2 * _t + 1
_INV = np.argsort(_PERM).astype(np.int32)


def _linear_norm(x, w, b, wp, bp):
    h = lax.dot_general(x, w, (((1,), (1,)), ((), ())),
                        preferred_element_type=_f32) + b
    hp = lax.dot_general(x, wp, (((1,), (1,)), ((), ())),
                         preferred_element_type=_f32) + bp
    nrm = jnp.sqrt(jnp.sum(h * h, axis=1, keepdims=True))
    scale = jnp.maximum(nrm, 1e-12)
    norm = h / scale
    normp = hp / scale
    selfmsg = jnp.sum(norm * norm, axis=1, keepdims=True) * h
    hi = scale.astype(jnp.bfloat16)
    lo = (scale - hi.astype(_f32)).astype(jnp.bfloat16)
    pad = jnp.zeros((norm.shape[0], TBL_W - C - 2), jnp.bfloat16)
    tbl = jnp.concatenate([normp.astype(jnp.bfloat16), hi, lo, pad], axis=1)
    return tbl, selfmsg


def _pre_body(x_ref, w_ref, b_ref, wp_ref, bp_ref, norm_ref, self_ref):
    norm, selfmsg = _linear_norm(x_ref[...], w_ref[...], b_ref[...],
                                 wp_ref[...], bp_ref[...])
    norm_ref[...] = norm
    self_ref[...] = selfmsg


def _tc_pre(x, w, b, wp, bp):
    return pl.pallas_call(
        _pre_body,
        grid=(N // BR,),
        in_specs=[pl.BlockSpec((BR, C), lambda i: (i, 0)),
                  pl.BlockSpec((C, C), lambda i: (0, 0)),
                  pl.BlockSpec((1, C), lambda i: (0, 0)),
                  pl.BlockSpec((C, C), lambda i: (0, 0)),
                  pl.BlockSpec((1, C), lambda i: (0, 0))],
        out_specs=[pl.BlockSpec((BR, TBL_W), lambda i: (i, 0)),
                   pl.BlockSpec((BR, C), lambda i: (i, 0))],
        out_shape=[jax.ShapeDtypeStruct((N, TBL_W), jnp.bfloat16),
                   jax.ShapeDtypeStruct((N, C), _f32)],
    )(x, w, b.reshape(1, C), wp, bp.reshape(1, C))


def _combine(a0, a1, c0, c1, selfmsg, bias):
    summed = a0 + a1 + selfmsg
    cnt = c0[:, 0:1] + c1[:, 0:1] + 1.0
    return summed / jnp.maximum(cnt, 1.0) + bias


def _mid_body(a0_ref, a1_ref, c0_ref, c1_ref, self_ref, bias_ref, w_ref,
              b_ref, wp_ref, bp_ref, norm_ref, self2_ref):
    x2 = _combine(a0_ref[0], a1_ref[0], c0_ref[0], c1_ref[0],
                  self_ref[...], bias_ref[...])
    x2 = jnp.maximum(x2, 0.0)
    norm, selfmsg = _linear_norm(x2, w_ref[...], b_ref[...],
                                 wp_ref[...], bp_ref[...])
    norm_ref[...] = norm
    self2_ref[...] = selfmsg


def _tc_mid(acc, cnt, selfmsg, bias, w, b, wp, bp):
    return pl.pallas_call(
        _mid_body,
        grid=(N // BR,),
        in_specs=[pl.BlockSpec((1, BR, C), lambda i: (0, i, 0)),
                  pl.BlockSpec((1, BR, C), lambda i: (1, i, 0)),
                  pl.BlockSpec((1, BR, 4), lambda i: (0, i, 0)),
                  pl.BlockSpec((1, BR, 4), lambda i: (1, i, 0)),
                  pl.BlockSpec((BR, C), lambda i: (i, 0)),
                  pl.BlockSpec((1, C), lambda i: (0, 0)),
                  pl.BlockSpec((C, C), lambda i: (0, 0)),
                  pl.BlockSpec((1, C), lambda i: (0, 0)),
                  pl.BlockSpec((C, C), lambda i: (0, 0)),
                  pl.BlockSpec((1, C), lambda i: (0, 0))],
        out_specs=[pl.BlockSpec((BR, TBL_W), lambda i: (i, 0)),
                   pl.BlockSpec((BR, C), lambda i: (i, 0))],
        out_shape=[jax.ShapeDtypeStruct((N, TBL_W), jnp.bfloat16),
                   jax.ShapeDtypeStruct((N, C), _f32)],
    )(acc, acc, cnt, cnt, selfmsg, bias.reshape(1, C), w,
      b.reshape(1, C), wp, bp.reshape(1, C))


def _final_body(a0_ref, a1_ref, c0_ref, c1_ref, self_ref, bias_ref,
                out_ref):
    h = _combine(a0_ref[0], a1_ref[0], c0_ref[0], c1_ref[0],
                 self_ref[...], bias_ref[...])
    m = jnp.max(h, axis=1, keepdims=True)
    z = h - m
    out_ref[...] = z - jnp.log(jnp.sum(jnp.exp(z), axis=1, keepdims=True))


def _tc_final(acc, cnt, selfmsg, bias):
    return pl.pallas_call(
        _final_body,
        grid=(N // BR,),
        in_specs=[pl.BlockSpec((1, BR, C), lambda i: (0, i, 0)),
                  pl.BlockSpec((1, BR, C), lambda i: (1, i, 0)),
                  pl.BlockSpec((1, BR, 4), lambda i: (0, i, 0)),
                  pl.BlockSpec((1, BR, 4), lambda i: (1, i, 0)),
                  pl.BlockSpec((BR, C), lambda i: (i, 0)),
                  pl.BlockSpec((1, C), lambda i: (0, 0))],
        out_specs=pl.BlockSpec((BR, C), lambda i: (i, 0)),
        out_shape=jax.ShapeDtypeStruct((N, C), _f32),
    )(acc, acc, cnt, cnt, selfmsg, bias.reshape(1, C))


def _sc_edge_body(edge_ref, norm_ref, zeros_ref, zeros_cnt_ref,
                  out_ref, cnt_out_ref,
                  sidx, didx, sdidx, didx2, nsrc, ndst, msg, cntbuf,
                  acc_sh, cnt_sh, isem, gsem, ssem):
    cc = lax.axis_index("c")
    ss = lax.axis_index("s")
    wid = cc * NS + ss

    pltpu.sync_copy(zeros_ref.at[pl.ds(ss * ROWS_PT, ROWS_PT)],
                    acc_sh.at[pl.ds(ss * ROWS_PT, ROWS_PT)])
    pltpu.sync_copy(zeros_cnt_ref.at[pl.ds(ss * CROWS_PT, CROWS_PT)],
                    cnt_sh.at[pl.ds(ss * CROWS_PT, CROWS_PT)])

    iota16 = lax.iota(_i32, L)
    ones16 = jnp.ones((L,), _f32)
    zeros16 = jnp.zeros((L,), _f32)
    for S in range(2):
        for g in range(GROUPS):
            e16 = iota16 + (g * L)
            for col in range(CNT_W):
                plsc.store_scatter(cntbuf[S],
                                   [e16, jnp.full((L,), col, _i32)], zeros16)

    plsc.subcore_barrier()

    base0 = wid * EPT

    def issue_idx(k, S):
        base = base0 + jnp.minimum(k * CH, EPT - CH)
        pltpu.async_copy(edge_ref.at[pl.ds(base, CH)], sidx[S], isem[S])
        pltpu.async_copy(edge_ref.at[pl.ds(E + base, CH)], didx[S], isem[S])

    def wait_idx(S):
        pltpu.make_async_copy(edge_ref.at[pl.ds(0, CH)], sidx[S],
                              isem[S]).wait()
        pltpu.make_async_copy(edge_ref.at[pl.ds(0, CH)], didx[S],
                              isem[S]).wait()

    def issue_gather(S):
        pltpu.async_copy(norm_ref.at[sidx[S]], nsrc[S], gsem[S])
        pltpu.async_copy(norm_ref.at[didx[S]], ndst[S], gsem[S])

    def wait_gather(S):
        pltpu.make_async_copy(norm_ref.at[sidx[S]], nsrc[S], gsem[S]).wait()
        pltpu.make_async_copy(norm_ref.at[didx[S]], ndst[S], gsem[S]).wait()

    def issue_scatter(S):
        pltpu.async_copy(msg[S], acc_sh.at[sdidx[S]], ssem[S], add=True)
        pltpu.async_copy(cntbuf[S], cnt_sh.at[didx2[S]], ssem[S], add=True)

    def wait_scatter(S):
        pltpu.make_async_copy(msg[S], acc_sh.at[sdidx[S]], ssem[S]).wait()
        pltpu.make_async_copy(cntbuf[S], cnt_sh.at[didx2[S]], ssem[S]).wait()

    def dst_save(S4, S2, dummies=()):
        for g in range(GROUPS):
            e16 = iota16 + (g * L)
            dst16 = didx[S4][pl.ds(g * L, L)]
            if g in dummies:
                dst16 = jnp.full((L,), N, _i32)
            sdidx[S2][pl.ds(g * L, L)] = dst16
            didx2[S2][pl.ds(g * L, L)] = lax.shift_right_logical(dst16, 2)
            q = jnp.bitwise_and(dst16, 3)
            for i in range(4):
                col = lax.shift_left(jnp.bitwise_xor(q, i), 2)
                plsc.store_scatter(cntbuf[S2], [e16, col],
                                   ones16 if i == 0 else zeros16)

    def dot_mul(S4, S2):
        def edge_body(e):
            sab = [plsc.unpack(nsrc[S4][e, pl.ds(32 * j, 32)],
                               format=plsc.PackFormat.INTERLEAVED)
                   for j in range(4)]
            dab = [plsc.unpack(ndst[S4][e, pl.ds(32 * j, 32)],
                               format=plsc.PackFormat.INTERLEAVED)
                   for j in range(4)]
            m = [sab[j][0] * dab[j][0] + sab[j][1] * dab[j][1]
                 for j in range(4)]
            ha, hb = plsc.unpack(nsrc[S4][e, pl.ds(C, 32)],
                                 format=plsc.PackFormat.INTERLEAVED)
            sc = ha[0] + hb[0]
            coef = jnp.sum((m[0] + m[1]) + (m[2] + m[3])) * sc
            for j in range(4):
                msg[S2][e, pl.ds(32 * j, L)] = sab[j][0] * coef
                msg[S2][e, pl.ds(32 * j + L, L)] = sab[j][1] * coef

        plsc.parallel_loop(0, CH, unroll=2)(edge_body)

    def step(k, S4, pf_idx, pf_gather, wait_scat, dummies=()):
        S2 = S4 % 2
        wait_gather(S4)
        if pf_gather:
            wait_idx((S4 + 2) % 4)
            issue_gather((S4 + 2) % 4)
        if wait_scat:
            wait_scatter(S2)
        dst_save(S4, S2, dummies)
        if pf_idx:
            issue_idx(k + 4, S4)
        dot_mul(S4, S2)
        issue_scatter(S2)

    for j in range(4):
        issue_idx(j, j)
    wait_idx(0)
    issue_gather(0)
    wait_idx(1)
    issue_gather(1)
    step(0, 0, True, True, False)
    step(1, 1, True, True, False)
    step(2, 2, True, True, True)
    step(3, 3, True, True, True)

    def quad(kk, carry):
        k0 = 4 * kk
        for j in range(4):
            step(k0 + j, j, True, True, True)
        return carry

    lax.fori_loop(1, (CHUNKS - 9) // 4 + 1, quad, 0)

    step(CHUNKS - 5, 0, True, True, True)
    step(CHUNKS - 4, 1, False, True, True)
    step(CHUNKS - 3, 2, False, True, True)
    step(CHUNKS - 2, 3, False, False, True)
    step(CHUNKS - 1, 0, False, False, True,
         dummies=tuple(range(GROUPS - (EPT - (CHUNKS - 1) * CH) // L)))
    wait_scatter(1)
    wait_scatter(0)

    plsc.subcore_barrier()
    pltpu.sync_copy(acc_sh.at[pl.ds(ss * ROWS_PT, ROWS_PT)],
                    out_ref.at[cc, pl.ds(ss * ROWS_PT, ROWS_PT)])
    pltpu.sync_copy(cnt_sh.at[pl.ds(ss * CROWS_PT, CROWS_PT)],
                    cnt_out_ref.at[cc, pl.ds(ss * CROWS_PT, CROWS_PT)])


_sc_mesh = plsc.VectorSubcoreMesh(core_axis_name="c", subcore_axis_name="s",
                                  num_cores=NC, num_subcores=NS)

_sc_edge = functools.partial(
    pl.kernel,
    out_type=(jax.ShapeDtypeStruct((NC, NPAD, C), _f32),
              jax.ShapeDtypeStruct((NC, CPAD, CNT_W), _f32)),
    mesh=_sc_mesh,
    compiler_params=pltpu.CompilerParams(needs_layout_passes=False,
                                         use_tc_tiling_on_sc=False),
    scratch_types=[
        [pltpu.VMEM((CH,), _i32)] * 4,
        [pltpu.VMEM((CH,), _i32)] * 4,
        [pltpu.VMEM((CH,), _i32)] * 2,
        [pltpu.VMEM((CH,), _i32)] * 2,
        [pltpu.VMEM((CH, TBL_W), jnp.bfloat16)] * 4,
        [pltpu.VMEM((CH, TBL_W), jnp.bfloat16)] * 4,
        [pltpu.VMEM((CH, C), _f32)] * 2,
        [pltpu.VMEM((CH, CNT_W), _f32)] * 2,
        pltpu.MemorySpace.VMEM_SHARED((NPAD, C), _f32),
        pltpu.MemorySpace.VMEM_SHARED((CPAD, CNT_W), _f32),
        [pltpu.SemaphoreType.DMA] * 4,
        [pltpu.SemaphoreType.DMA] * 4,
        [pltpu.SemaphoreType.DMA] * 2,
    ],
)(_sc_edge_body)


def kernel(x, edge_index, W1, b1, bias1, W2, b2, bias2):
    zeros = jnp.zeros((NPAD, C), _f32)
    zeros_cnt = jnp.zeros((CPAD, CNT_W), _f32)
    edge_flat = edge_index.reshape(2 * E)
    inv = jnp.asarray(_INV)
    W1p, b1p = jnp.take(W1, inv, axis=0), jnp.take(b1, inv, axis=0)
    W2p, b2p = jnp.take(W2, inv, axis=0), jnp.take(b2, inv, axis=0)
    norm1, self1 = _tc_pre(x, W1, b1, W1p, b1p)
    acc1, cnt1 = _sc_edge(edge_flat, norm1, zeros, zeros_cnt)
    cnt1 = cnt1.reshape(NC, CPAD * 4, 4)
    norm2, self2 = _tc_mid(acc1, cnt1, self1, bias1, W2, b2, W2p, b2p)
    acc2, cnt2 = _sc_edge(edge_flat, norm2, zeros, zeros_cnt)
    cnt2 = cnt2.reshape(NC, CPAD * 4, 4)
    return _tc_final(acc2, cnt2, self2, bias2)


# speedup vs baseline: 1.1741x; 1.0137x over previous
import functools

import jax
import jax.numpy as jnp
import numpy as np
from jax import lax
from jax.experimental import pallas as pl
from jax.experimental.pallas import tpu as pltpu
from jax.experimental.pallas import tpu_sc as plsc

N = 10000
C = 128
E = 320000
NC = 2
NS = 16
NW = NC * NS
L = 16
EPT = E // NW
CH = 48
GROUPS = CH // L
CHUNKS = -(-EPT // CH)
EXT_W = C + 16
TBL_W = 160
ROWS_PT = 632
NPAD = ROWS_PT * NS
CNT_W = 16
CROWS_PT = 160
CPAD = CROWS_PT * NS

_f32 = jnp.float32
_i32 = jnp.int32

BR = 2000

_PERM = np.empty((C,), np.int32)
for _j in range(C // 32):
    for _t in range(16):
        _PERM[32 * _j + _t] = 32 * _j + 2 * _t
        _PERM[32 * _j + 16 + _t] = 32 * _j + 2 * _t + 1
_INV = np.argsort(_PERM).astype(np.int32)


def _linear_norm(x, w, b, wp, bp):
    h = lax.dot_general(x, w, (((1,), (1,)), ((), ())),
                        preferred_element_type=_f32) + b
    hp = lax.dot_general(x, wp, (((1,), (1,)), ((), ())),
                         preferred_element_type=_f32) + bp
    nrm = jnp.sqrt(jnp.sum(h * h, axis=1, keepdims=True))
    scale = jnp.maximum(nrm, 1e-12)
    norm = h / scale
    normp = hp / scale
    selfmsg = jnp.sum(norm * norm, axis=1, keepdims=True) * h
    hi = scale.astype(jnp.bfloat16)
    lo = (scale - hi.astype(_f32)).astype(jnp.bfloat16)
    pad = jnp.zeros((norm.shape[0], TBL_W - C - 2), jnp.bfloat16)
    tbl = jnp.concatenate([normp.astype(jnp.bfloat16), hi, lo, pad], axis=1)
    return tbl, selfmsg


def _pre_body(x_ref, w_ref, b_ref, wp_ref, bp_ref, norm_ref, self_ref):
    norm, selfmsg = _linear_norm(x_ref[...], w_ref[...], b_ref[...],
                                 wp_ref[...], bp_ref[...])
    norm_ref[...] = norm
    self_ref[...] = selfmsg


def _tc_pre(x, w, b, wp, bp):
    return pl.pallas_call(
        _pre_body,
        grid=(N // BR,),
        in_specs=[pl.BlockSpec((BR, C), lambda i: (i, 0)),
                  pl.BlockSpec((C, C), lambda i: (0, 0)),
                  pl.BlockSpec((1, C), lambda i: (0, 0)),
                  pl.BlockSpec((C, C), lambda i: (0, 0)),
                  pl.BlockSpec((1, C), lambda i: (0, 0))],
        out_specs=[pl.BlockSpec((BR, TBL_W), lambda i: (i, 0)),
                   pl.BlockSpec((BR, C), lambda i: (i, 0))],
        out_shape=[jax.ShapeDtypeStruct((N, TBL_W), jnp.bfloat16),
                   jax.ShapeDtypeStruct((N, C), _f32)],
    )(x, w, b.reshape(1, C), wp, bp.reshape(1, C))


def _combine(a0, a1, c0, c1, selfmsg, bias):
    summed = a0 + a1 + selfmsg
    cnt = c0[:, 0:1] + c1[:, 0:1] + 1.0
    return summed / jnp.maximum(cnt, 1.0) + bias


def _mid_body(a0_ref, a1_ref, c0_ref, c1_ref, self_ref, bias_ref, w_ref,
              b_ref, wp_ref, bp_ref, norm_ref, self2_ref):
    x2 = _combine(a0_ref[0], a1_ref[0], c0_ref[0], c1_ref[0],
                  self_ref[...], bias_ref[...])
    x2 = jnp.maximum(x2, 0.0)
    norm, selfmsg = _linear_norm(x2, w_ref[...], b_ref[...],
                                 wp_ref[...], bp_ref[...])
    norm_ref[...] = norm
    self2_ref[...] = selfmsg


def _tc_mid(acc, cnt, selfmsg, bias, w, b, wp, bp):
    return pl.pallas_call(
        _mid_body,
        grid=(N // BR,),
        in_specs=[pl.BlockSpec((1, BR, C), lambda i: (0, i, 0)),
                  pl.BlockSpec((1, BR, C), lambda i: (1, i, 0)),
                  pl.BlockSpec((1, BR, 4), lambda i: (0, i, 0)),
                  pl.BlockSpec((1, BR, 4), lambda i: (1, i, 0)),
                  pl.BlockSpec((BR, C), lambda i: (i, 0)),
                  pl.BlockSpec((1, C), lambda i: (0, 0)),
                  pl.BlockSpec((C, C), lambda i: (0, 0)),
                  pl.BlockSpec((1, C), lambda i: (0, 0)),
                  pl.BlockSpec((C, C), lambda i: (0, 0)),
                  pl.BlockSpec((1, C), lambda i: (0, 0))],
        out_specs=[pl.BlockSpec((BR, TBL_W), lambda i: (i, 0)),
                   pl.BlockSpec((BR, C), lambda i: (i, 0))],
        out_shape=[jax.ShapeDtypeStruct((N, TBL_W), jnp.bfloat16),
                   jax.ShapeDtypeStruct((N, C), _f32)],
    )(acc, acc, cnt, cnt, selfmsg, bias.reshape(1, C), w,
      b.reshape(1, C), wp, bp.reshape(1, C))


def _final_body(a0_ref, a1_ref, c0_ref, c1_ref, self_ref, bias_ref,
                out_ref):
    h = _combine(a0_ref[0], a1_ref[0], c0_ref[0], c1_ref[0],
                 self_ref[...], bias_ref[...])
    m = jnp.max(h, axis=1, keepdims=True)
    z = h - m
    out_ref[...] = z - jnp.log(jnp.sum(jnp.exp(z), axis=1, keepdims=True))


def _tc_final(acc, cnt, selfmsg, bias):
    return pl.pallas_call(
        _final_body,
        grid=(N // BR,),
        in_specs=[pl.BlockSpec((1, BR, C), lambda i: (0, i, 0)),
                  pl.BlockSpec((1, BR, C), lambda i: (1, i, 0)),
                  pl.BlockSpec((1, BR, 4), lambda i: (0, i, 0)),
                  pl.BlockSpec((1, BR, 4), lambda i: (1, i, 0)),
                  pl.BlockSpec((BR, C), lambda i: (i, 0)),
                  pl.BlockSpec((1, C), lambda i: (0, 0))],
        out_specs=pl.BlockSpec((BR, C), lambda i: (i, 0)),
        out_shape=jax.ShapeDtypeStruct((N, C), _f32),
    )(acc, acc, cnt, cnt, selfmsg, bias.reshape(1, C))


def _sc_edge_body(edge_ref, norm_ref, zeros_ref, zeros_cnt_ref,
                  out_ref, cnt_out_ref,
                  sidx, didx, sdidx, didx2, nsrc, ndst, msg, cntbuf,
                  acc_sh, cnt_sh, isem, gsem, ssem):
    cc = lax.axis_index("c")
    ss = lax.axis_index("s")
    wid = cc * NS + ss

    pltpu.sync_copy(zeros_ref.at[pl.ds(ss * ROWS_PT, ROWS_PT)],
                    acc_sh.at[pl.ds(ss * ROWS_PT, ROWS_PT)])
    pltpu.sync_copy(zeros_cnt_ref.at[pl.ds(ss * CROWS_PT, CROWS_PT)],
                    cnt_sh.at[pl.ds(ss * CROWS_PT, CROWS_PT)])

    iota16 = lax.iota(_i32, L)
    ones16 = jnp.ones((L,), _f32)
    zeros16 = jnp.zeros((L,), _f32)
    for S in range(2):
        for g in range(GROUPS):
            e16 = iota16 + (g * L)
            for col in range(CNT_W):
                plsc.store_scatter(cntbuf[S],
                                   [e16, jnp.full((L,), col, _i32)], zeros16)

    plsc.subcore_barrier()

    base0 = wid * EPT

    def issue_idx(k, S):
        base = base0 + jnp.minimum(k * CH, EPT - CH)
        pltpu.async_copy(edge_ref.at[pl.ds(base, CH)], sidx[S], isem[S])
        pltpu.async_copy(edge_ref.at[pl.ds(E + base, CH)], didx[S], isem[S])

    def wait_idx(S):
        pltpu.make_async_copy(edge_ref.at[pl.ds(0, CH)], sidx[S],
                              isem[S]).wait()
        pltpu.make_async_copy(edge_ref.at[pl.ds(0, CH)], didx[S],
                              isem[S]).wait()

    def issue_gather(S):
        pltpu.async_copy(norm_ref.at[sidx[S]], nsrc[S], gsem[S])
        pltpu.async_copy(norm_ref.at[didx[S]], ndst[S], gsem[S])

    def wait_gather(S):
        pltpu.make_async_copy(norm_ref.at[sidx[S]], nsrc[S], gsem[S]).wait()
        pltpu.make_async_copy(norm_ref.at[didx[S]], ndst[S], gsem[S]).wait()

    def issue_scatter(S):
        pltpu.async_copy(msg[S], acc_sh.at[sdidx[S]], ssem[S], add=True)
        pltpu.async_copy(cntbuf[S], cnt_sh.at[didx2[S]], ssem[S], add=True)

    def wait_scatter(S):
        pltpu.make_async_copy(msg[S], acc_sh.at[sdidx[S]], ssem[S]).wait()
        pltpu.make_async_copy(cntbuf[S], cnt_sh.at[didx2[S]], ssem[S]).wait()

    def dst_save(S4, S2, dummies=()):
        for g in range(GROUPS):
            e16 = iota16 + (g * L)
            dst16 = didx[S4][pl.ds(g * L, L)]
            if g in dummies:
                dst16 = jnp.full((L,), N, _i32)
            sdidx[S2][pl.ds(g * L, L)] = dst16
            didx2[S2][pl.ds(g * L, L)] = lax.shift_right_logical(dst16, 2)
            q = jnp.bitwise_and(dst16, 3)
            for i in range(4):
                col = lax.shift_left(jnp.bitwise_xor(q, i), 2)
                plsc.store_scatter(cntbuf[S2], [e16, col],
                                   ones16 if i == 0 else zeros16)

    def dot_mul(S4, S2):
        def edge_body(e):
            sab = [plsc.unpack(nsrc[S4][e, pl.ds(32 * j, 32)],
                               format=plsc.PackFormat.INTERLEAVED)
                   for j in range(4)]
            dab = [plsc.unpack(ndst[S4][e, pl.ds(32 * j, 32)],
                               format=plsc.PackFormat.INTERLEAVED)
                   for j in range(4)]
            m = [sab[j][0] * dab[j][0] + sab[j][1] * dab[j][1]
                 for j in range(4)]
            ha, hb = plsc.unpack(nsrc[S4][e, pl.ds(C, 32)],
                                 format=plsc.PackFormat.INTERLEAVED)
            sc = ha[0] + hb[0]
            coef = jnp.sum((m[0] + m[1]) + (m[2] + m[3])) * sc
            for j in range(4):
                msg[S2][e, pl.ds(32 * j, L)] = sab[j][0] * coef
                msg[S2][e, pl.ds(32 * j + L, L)] = sab[j][1] * coef

        plsc.parallel_loop(0, CH, unroll=2)(edge_body)

    def step(k, S4, pf_idx, pf_gather, wait_scat, dummies=()):
        S2 = S4 % 2
        wait_gather(S4)
        if pf_gather:
            wait_idx((S4 + 2) % 4)
            issue_gather((S4 + 2) % 4)
        if wait_scat:
            wait_scatter(S2)
        dst_save(S4, S2, dummies)
        if pf_idx:
            issue_idx(k + 4, S4)
        dot_mul(S4, S2)
        issue_scatter(S2)

    for j in range(4):
        issue_idx(j, j)
    wait_idx(0)
    issue_gather(0)
    wait_idx(1)
    issue_gather(1)
    step(0, 0, True, True, False)
    step(1, 1, True, True, False)
    step(2, 2, True, True, True)
    step(3, 3, True, True, True)

    def quad(kk, carry):
        k0 = 4 * kk
        for j in range(4):
            step(k0 + j, j, True, True, True)
        return carry

    lax.fori_loop(1, (CHUNKS - 9) // 4 + 1, quad, 0)

    step(CHUNKS - 5, 0, True, True, True)
    step(CHUNKS - 4, 1, False, True, True)
    step(CHUNKS - 3, 2, False, True, True)
    step(CHUNKS - 2, 3, False, False, True)
    step(CHUNKS - 1, 0, False, False, True,
         dummies=tuple(range(GROUPS - (EPT - (CHUNKS - 1) * CH) // L)))
    wait_scatter(1)
    wait_scatter(0)

    plsc.subcore_barrier()
    pltpu.sync_copy(acc_sh.at[pl.ds(ss * ROWS_PT, ROWS_PT)],
                    out_ref.at[cc, pl.ds(ss * ROWS_PT, ROWS_PT)])
    pltpu.sync_copy(cnt_sh.at[pl.ds(ss * CROWS_PT, CROWS_PT)],
                    cnt_out_ref.at[cc, pl.ds(ss * CROWS_PT, CROWS_PT)])


_sc_mesh = plsc.VectorSubcoreMesh(core_axis_name="c", subcore_axis_name="s",
                                  num_cores=NC, num_subcores=NS)

_sc_edge = functools.partial(
    pl.kernel,
    out_type=(jax.ShapeDtypeStruct((NC, NPAD, C), _f32),
              jax.ShapeDtypeStruct((NC, CPAD, CNT_W), _f32)),
    mesh=_sc_mesh,
    compiler_params=pltpu.CompilerParams(needs_layout_passes=False,
                                         use_tc_tiling_on_sc=False),
    scratch_types=[
        [pltpu.VMEM((CH,), _i32)] * 4,
        [pltpu.VMEM((CH,), _i32)] * 4,
        [pltpu.VMEM((CH,), _i32)] * 2,
        [pltpu.VMEM((CH,), _i32)] * 2,
        [pltpu.VMEM((CH, TBL_W), jnp.bfloat16)] * 4,
        [pltpu.VMEM((CH, TBL_W), jnp.bfloat16)] * 4,
        [pltpu.VMEM((CH, C), _f32)] * 2,
        [pltpu.VMEM((CH, CNT_W), _f32)] * 2,
        pltpu.MemorySpace.VMEM_SHARED((NPAD, C), _f32),
        pltpu.MemorySpace.VMEM_SHARED((CPAD, CNT_W), _f32),
        [pltpu.SemaphoreType.DMA] * 4,
        [pltpu.SemaphoreType.DMA] * 4,
        [pltpu.SemaphoreType.DMA] * 2,
    ],
)(_sc_edge_body)


def kernel(x, edge_index, W1, b1, bias1, W2, b2, bias2):
    zeros = jnp.zeros((NPAD, C), _f32)
    zeros_cnt = jnp.zeros((CPAD, CNT_W), _f32)
    edge_flat = edge_index.reshape(2 * E)
    inv = jnp.asarray(_INV)
    W1p, b1p = jnp.take(W1, inv, axis=0), jnp.take(b1, inv, axis=0)
    W2p, b2p = jnp.take(W2, inv, axis=0), jnp.take(b2, inv, axis=0)
    norm1, self1 = _tc_pre(x, W1, b1, W1p, b1p)
    acc1, cnt1 = _sc_edge(edge_flat, norm1, zeros, zeros_cnt)
    cnt1 = cnt1.reshape(NC, CPAD * 4, 4)
    norm2, self2 = _tc_mid(acc1, cnt1, self1, bias1, W2, b2, W2p, b2p)
    acc2, cnt2 = _sc_edge(edge_flat, norm2, zeros, zeros_cnt)
    cnt2 = cnt2.reshape(NC, CPAD * 4, 4)
    return _tc_final(acc2, cnt2, self2, bias2)
